# baseline (speedup 1.0000x reference)
import functools

import jax
import jax.numpy as jnp
from jax import lax
from jax.experimental import pallas as pl
from jax.experimental.pallas import tpu as pltpu
from jax.experimental.pallas import tpu_sc as plsc

_BBOX = 1000
_VOCAB = _BBOX + 3
_HID = 128
_L = 16
_C = 512
_GB = 128
_NB = _C // _GB


def _build(n_rows):
    info = plsc.get_sparse_core_info()
    nc, ns = info.num_cores, info.num_subcores
    nw = nc * ns
    rows_pw = n_rows // nw
    n_chunks = rows_pw // _C
    mesh = plsc.VectorSubcoreMesh(core_axis_name="c", subcore_axis_name="s")

    @functools.partial(
        pl.kernel,
        mesh=mesh,
        out_type=jax.ShapeDtypeStruct((n_rows, _HID), jnp.float32),
        scratch_types=[
            pltpu.VMEM((6, _C), jnp.int32),
            pltpu.VMEM((14, _NB, _GB), jnp.int32),
            pltpu.VMEM((_C, _HID), jnp.float32),
            pltpu.SemaphoreType.DMA,
        ],
    )
    def k(comps_hbm, ctab_hbm, out_hbm, cv, idxv, acc, gsem):
        wid = lax.axis_index("s") * nc + lax.axis_index("c")
        base0 = wid * rows_pw

        def chunk_body(ci, carry):
            base = base0 + ci * _C
            pltpu.sync_copy(comps_hbm.at[:, pl.ds(base, _C)], cv)

            def zero_body(r, carry2):
                for j in range(_HID // _L):
                    acc[r, pl.ds(j * _L, _L)] = jnp.zeros((_L,), jnp.float32)
                return carry2

            def batch_body(b, carry2):
                def idx_body(i, carry3):
                    s = b * _GB + i * _L
                    cx = cv[0, pl.ds(s, _L)]
                    cy = cv[1, pl.ds(s, _L)]
                    w = cv[2, pl.ds(s, _L)]
                    h = cv[3, pl.ds(s, _L)]
                    xs = cv[4, pl.ds(s, _L)]
                    ys = cv[5, pl.ds(s, _L)]
                    xa = ((xs - _BBOX // 2).astype(jnp.float32) * 0.5
                          ).astype(jnp.int32)
                    ya = ((ys - _BBOX // 2).astype(jnp.float32) * 0.5
                          ).astype(jnp.int32)
                    w2 = lax.shift_right_arithmetic(w, 1)
                    h2 = lax.shift_right_arithmetic(h, 1)

                    def clip(v):
                        return jnp.minimum(jnp.maximum(v, 0), _BBOX)

                    o = i * _L
                    idxv[0, b, pl.ds(o, _L)] = w
                    idxv[1, b, pl.ds(o, _L)] = h + _VOCAB
                    idxv[2, b, pl.ds(o, _L)] = cx + 2 * _VOCAB
                    idxv[3, b, pl.ds(o, _L)] = cy + 3 * _VOCAB
                    idxv[4, b, pl.ds(o, _L)] = xs + 4 * _VOCAB
                    idxv[5, b, pl.ds(o, _L)] = ys + 5 * _VOCAB
                    x1 = clip(cx - w2 - xa)
                    y1 = clip(cy - h2 - ya)
                    x2 = clip(cx + w2 - xa)
                    y2 = clip(cy + h2 + ya)
                    x3 = clip(cx + w2 + xa)
                    x4 = clip(cx - w2 + xa)
                    idxv[6, b, pl.ds(o, _L)] = x1 + 6 * _VOCAB
                    idxv[7, b, pl.ds(o, _L)] = y1 + 7 * _VOCAB
                    idxv[8, b, pl.ds(o, _L)] = x2 + 8 * _VOCAB
                    idxv[9, b, pl.ds(o, _L)] = y2 + 9 * _VOCAB
                    idxv[10, b, pl.ds(o, _L)] = x3 + 10 * _VOCAB
                    idxv[11, b, pl.ds(o, _L)] = y2 + 11 * _VOCAB
                    idxv[12, b, pl.ds(o, _L)] = x4 + 12 * _VOCAB
                    idxv[13, b, pl.ds(o, _L)] = y1 + 13 * _VOCAB
                    return carry3

                lax.fori_loop(0, _GB // _L, idx_body, 0)
                return carry2

            lax.fori_loop(0, _C, zero_body, 0)
            lax.fori_loop(0, _NB, batch_body, 0)

            def fire_body(t, carry2):
                def fire_b(b, carry3):
                    pltpu.async_copy(
                        ctab_hbm.at[idxv.at[t, b]],
                        acc.at[pl.ds(b * _GB, _GB)],
                        gsem, add=True)
                    return carry3
                lax.fori_loop(0, _NB, fire_b, 0)
                return carry2

            def drain_body(t, carry2):
                def drain_b(b, carry3):
                    pltpu.make_async_copy(
                        ctab_hbm.at[idxv.at[t, b]],
                        acc.at[pl.ds(b * _GB, _GB)],
                        gsem).wait()
                    return carry3
                lax.fori_loop(0, _NB, drain_b, 0)
                return carry2

            lax.fori_loop(0, 14, fire_body, 0)
            lax.fori_loop(0, 14, drain_body, 0)
            pltpu.sync_copy(acc, out_hbm.at[pl.ds(base, _C)])
            return carry

        lax.fori_loop(0, n_chunks, chunk_body, 0)

    return k


def kernel(boxes, tables):
    b, s, _ = boxes.shape
    n_rows = b * s
    comps = boxes.astype(jnp.int32).reshape(n_rows, 6).T
    ctab = tables.reshape(14 * _VOCAB, _HID)
    out = _build(n_rows)(comps, ctab)
    return out.reshape(b, s, _HID)


# speedup vs baseline: 7.2276x; 7.2276x over previous
import functools

import jax
import jax.numpy as jnp
from jax import lax
from jax.experimental import pallas as pl
from jax.experimental.pallas import tpu as pltpu
from jax.experimental.pallas import tpu_sc as plsc

_BBOX = 1000
_VOCAB = _BBOX + 3
_HID = 128
_L = 16
_C = 128
_TPAD = 1024
_NT = 12

_SOURCES = [(0,), (1,), (2,), (3,), (4,), (5,), (6,), (7, 13), (8,),
            (9, 11), (10,), (12,)]


def _build(n_rows):
    info = plsc.get_sparse_core_info()
    nc, ns = info.num_cores, info.num_subcores
    nw = nc * ns
    rows_pw = n_rows // nw
    n_chunks = rows_pw // _C
    srows = _TPAD // ns
    mesh = plsc.VectorSubcoreMesh(core_axis_name="c", subcore_axis_name="s")

    @functools.partial(
        pl.kernel,
        mesh=mesh,
        out_type=jax.ShapeDtypeStruct((n_rows, _HID), jnp.float32),
        scratch_types=[
            pltpu.VMEM((6, _C), jnp.int32),
            pltpu.VMEM((_NT, 1, _C), jnp.int32),
            pltpu.VMEM((_C, _HID), jnp.float32),
            pltpu.VMEM_SHARED((_NT * _TPAD, _HID), jnp.float32),
            pltpu.SemaphoreType.DMA,
        ],
    )
    def k(comps_hbm, ctab_hbm, out_hbm, cv, idxv, acc, stab, gsem):
        sid = lax.axis_index("s")
        wid = sid * nc + lax.axis_index("c")
        base0 = wid * rows_pw

        soff = sid * srows
        for e, srcs in enumerate(_SOURCES):
            dst = stab.at[pl.ds(e * _TPAD + soff, srows)]
            if len(srcs) == 1:
                pltpu.sync_copy(
                    ctab_hbm.at[pl.ds(srcs[0] * _TPAD + soff, srows)], dst)
            else:
                pltpu.sync_copy(
                    ctab_hbm.at[pl.ds(srcs[0] * _TPAD + soff, srows)],
                    acc.at[pl.ds(0, srows)])
                pltpu.sync_copy(
                    ctab_hbm.at[pl.ds(srcs[1] * _TPAD + soff, srows)],
                    acc.at[pl.ds(srows, srows)])

                def sum_body(r, carry):
                    for j in range(_HID // _L):
                        s = pl.ds(j * _L, _L)
                        acc[r, s] = acc[r, s] + acc[srows + r, s]
                    return carry

                lax.fori_loop(0, srows, sum_body, 0)
                pltpu.sync_copy(acc.at[pl.ds(0, srows)], dst)
        plsc.subcore_barrier()

        def chunk_body(ci, carry):
            base = base0 + ci * _C
            pltpu.sync_copy(comps_hbm.at[:, pl.ds(base, _C)], cv)

            def zero_body(r, carry2):
                for j in range(_HID // _L):
                    acc[r, pl.ds(j * _L, _L)] = jnp.zeros((_L,), jnp.float32)
                return carry2

            def idx_body(i, carry2):
                s = i * _L
                cx = cv[0, pl.ds(s, _L)]
                cy = cv[1, pl.ds(s, _L)]
                w = cv[2, pl.ds(s, _L)]
                h = cv[3, pl.ds(s, _L)]
                xs = cv[4, pl.ds(s, _L)]
                ys = cv[5, pl.ds(s, _L)]
                xa = ((xs - _BBOX // 2).astype(jnp.float32) * 0.5
                      ).astype(jnp.int32)
                ya = ((ys - _BBOX // 2).astype(jnp.float32) * 0.5
                      ).astype(jnp.int32)
                w2 = lax.shift_right_arithmetic(w, 1)
                h2 = lax.shift_right_arithmetic(h, 1)

                def clip(v):
                    return jnp.minimum(jnp.maximum(v, 0), _BBOX)

                sl = pl.ds(s, _L)
                idxv[0, 0, sl] = w
                idxv[1, 0, sl] = h + _TPAD
                idxv[2, 0, sl] = cx + 2 * _TPAD
                idxv[3, 0, sl] = cy + 3 * _TPAD
                idxv[4, 0, sl] = xs + 4 * _TPAD
                idxv[5, 0, sl] = ys + 5 * _TPAD
                idxv[6, 0, sl] = clip(cx - w2 - xa) + 6 * _TPAD
                idxv[7, 0, sl] = clip(cy - h2 - ya) + 7 * _TPAD
                idxv[8, 0, sl] = clip(cx + w2 - xa) + 8 * _TPAD
                idxv[9, 0, sl] = clip(cy + h2 + ya) + 9 * _TPAD
                idxv[10, 0, sl] = clip(cx + w2 + xa) + 10 * _TPAD
                idxv[11, 0, sl] = clip(cx - w2 + xa) + 11 * _TPAD
                return carry2

            lax.fori_loop(0, _C, zero_body, 0)
            lax.fori_loop(0, _C // _L, idx_body, 0)

            def fire_body(t, carry2):
                pltpu.async_copy(stab.at[idxv.at[t, 0]], acc, gsem, add=True)
                return carry2

            def drain_body(t, carry2):
                pltpu.make_async_copy(stab.at[idxv.at[t, 0]], acc, gsem
                                      ).wait()
                return carry2

            lax.fori_loop(0, _NT, fire_body, 0)
            lax.fori_loop(0, _NT, drain_body, 0)
            pltpu.sync_copy(acc, out_hbm.at[pl.ds(base, _C)])
            return carry

        lax.fori_loop(0, n_chunks, chunk_body, 0)

    return k


def kernel(boxes, tables):
    b, s, _ = boxes.shape
    n_rows = b * s
    comps = boxes.astype(jnp.int32).reshape(n_rows, 6).T
    ctab = jnp.pad(tables, ((0, 0), (0, _TPAD - _VOCAB), (0, 0))
                   ).reshape(14 * _TPAD, _HID)
    out = _build(n_rows)(comps, ctab)
    return out.reshape(b, s, _HID)
